# baseline (speedup 1.0000x reference)
import functools

import jax
import jax.numpy as jnp
from jax import lax
from jax.experimental import pallas as pl
from jax.experimental.pallas import tpu as pltpu
from jax.experimental.pallas import tpu_sc as plsc

N = 10000
E = 320000
FD = 128
LD = 16
HD = 128
OD = 40

NC, NS = 2, 16
NW = NC * NS

DCH = 128
DNCH = 80
EPW = DCH * DNCH
EPAD = EPW * NW
RPS = 632
NPAD = RPS * NS

CH = 32
NCH = EPAD // NS // CH
NHALF = N // 2
ARPS = 320
APAD = ARPS * NS
HPS = 632

_mesh = plsc.VectorSubcoreMesh(core_axis_name="c", subcore_axis_name="s",
                               num_cores=NC, num_subcores=NS)


@functools.partial(
    pl.kernel,
    out_type=jax.ShapeDtypeStruct((NC * NPAD,), jnp.float32),
    mesh=_mesh,
    scratch_types=[
        pltpu.VMEM((DNCH, DCH), jnp.int32),
        pltpu.VMEM((DCH,), jnp.float32),
        pltpu.VMEM((RPS,), jnp.float32),
        pltpu.VMEM_SHARED((NPAD,), jnp.float32),
    ],
)
def _deg_kernel(dst_hbm, zeros1_hbm, ones_hbm, out_hbm, idx_v, ones_v,
                stage_v, deg_sh):
    cid = lax.axis_index("c")
    sid = lax.axis_index("s")
    wid = sid * NC + cid
    r0 = sid * RPS
    pltpu.sync_copy(zeros1_hbm, stage_v)
    pltpu.sync_copy(stage_v, deg_sh.at[pl.ds(r0, RPS)])
    pltpu.sync_copy(ones_hbm, ones_v)
    pltpu.sync_copy(dst_hbm.at[wid], idx_v)
    plsc.subcore_barrier()

    def body(j, carry):
        pltpu.sync_copy(ones_v, deg_sh.at[idx_v.at[j]], add=True)
        return carry

    lax.fori_loop(0, DNCH, body, 0)
    plsc.subcore_barrier()
    pltpu.sync_copy(deg_sh.at[pl.ds(r0, RPS)], stage_v)
    pltpu.sync_copy(stage_v, out_hbm.at[pl.ds(cid * NPAD + r0, RPS)])


@functools.partial(
    pl.kernel,
    out_type=jax.ShapeDtypeStruct((NC, APAD, HD), jnp.float32),
    mesh=_mesh,
    scratch_types=[
        pltpu.VMEM((1, CH), jnp.int32),
        pltpu.VMEM((1, CH), jnp.int32),
        pltpu.VMEM((1, CH), jnp.int32),
        pltpu.VMEM((1, CH), jnp.int32),
        pltpu.VMEM((CH, HD), jnp.float32),
        pltpu.VMEM((CH, HD), jnp.float32),
        pltpu.VMEM_SHARED((NPAD, HD), jnp.float32),
        pltpu.VMEM_SHARED((APAD, HD), jnp.float32),
        pltpu.SemaphoreType.DMA,
        pltpu.SemaphoreType.DMA,
        pltpu.SemaphoreType.DMA,
        pltpu.SemaphoreType.DMA,
        pltpu.SemaphoreType.DMA,
        pltpu.SemaphoreType.DMA,
    ],
)
def _agg_kernel(hp_hbm, src_hbm, dst_hbm, zeros2_hbm, out_hbm,
                srcc0, srcc1, dstc0, dstc1, rows0, rows1, hp_sh, agg_sh,
                si0, si1, di0, di1, sg0, sg1):
    cid = lax.axis_index("c")
    sid = lax.axis_index("s")
    r0 = sid * ARPS
    t0 = sid * HPS
    pltpu.sync_copy(hp_hbm.at[pl.ds(t0, HPS)], hp_sh.at[pl.ds(t0, HPS)])
    pltpu.sync_copy(zeros2_hbm, agg_sh.at[pl.ds(r0, DCH)])
    pltpu.sync_copy(zeros2_hbm, agg_sh.at[pl.ds(r0 + DCH, DCH)])
    pltpu.sync_copy(zeros2_hbm.at[pl.ds(0, ARPS - 2 * DCH)],
                    agg_sh.at[pl.ds(r0 + 2 * DCH, ARPS - 2 * DCH)])
    plsc.subcore_barrier()

    pltpu.async_copy(src_hbm.at[sid, 0], srcc0.at[0], si0)
    pltpu.async_copy(dst_hbm.at[cid, sid, 0], dstc0.at[0], di0)
    pltpu.async_copy(src_hbm.at[sid, 1], srcc1.at[0], si1)
    pltpu.async_copy(dst_hbm.at[cid, sid, 1], dstc1.at[0], di1)

    def body(t, carry):
        j = 2 * t
        pltpu.make_async_copy(src_hbm.at[sid, j], srcc0.at[0], si0).wait()
        pltpu.async_copy(hp_sh.at[srcc0.at[0]], rows0, sg0)
        pltpu.make_async_copy(src_hbm.at[sid, j + 1], srcc1.at[0], si1).wait()
        pltpu.async_copy(hp_sh.at[srcc1.at[0]], rows1, sg1)

        pltpu.make_async_copy(hp_sh.at[srcc0.at[0]], rows0, sg0).wait()
        pltpu.make_async_copy(dst_hbm.at[cid, sid, j], dstc0.at[0], di0).wait()
        pltpu.sync_copy(rows0, agg_sh.at[dstc0.at[0]], add=True)

        @pl.when(j + 2 < NCH)
        def _():
            pltpu.async_copy(src_hbm.at[sid, j + 2], srcc0.at[0], si0)
            pltpu.async_copy(dst_hbm.at[cid, sid, j + 2], dstc0.at[0], di0)

        pltpu.make_async_copy(hp_sh.at[srcc1.at[0]], rows1, sg1).wait()
        pltpu.make_async_copy(dst_hbm.at[cid, sid, j + 1], dstc1.at[0],
                              di1).wait()
        pltpu.sync_copy(rows1, agg_sh.at[dstc1.at[0]], add=True)

        @pl.when(j + 3 < NCH)
        def _():
            pltpu.async_copy(src_hbm.at[sid, j + 3], srcc1.at[0], si1)
            pltpu.async_copy(dst_hbm.at[cid, sid, j + 3], dstc1.at[0], di1)

        return carry

    lax.fori_loop(0, NCH // 2, body, 0)
    plsc.subcore_barrier()
    pltpu.sync_copy(agg_sh.at[pl.ds(r0, ARPS)],
                    out_hbm.at[cid, pl.ds(r0, ARPS)])


RB = 1000


def _tc1_body(x_ref, xl_ref, deg_ref, wg_ref, wm_ref, bm_ref,
              hp_ref, mlp_ref, dinv_ref):
    deg = deg_ref[...]
    degs = deg[:, 0] + deg[:, 1] + 1.0
    dinv = lax.rsqrt(degs)
    x = x_ref[...]
    h = jnp.dot(x, wg_ref[...], preferred_element_type=jnp.float32)
    hp_ref[...] = h * dinv[:, None]
    m = jnp.dot(x, wm_ref[0:FD], preferred_element_type=jnp.float32)
    m = m + jnp.dot(xl_ref[...], wm_ref[FD:FD + LD],
                    preferred_element_type=jnp.float32)
    mlp_ref[...] = jnp.maximum(m + bm_ref[...], 0.0)
    dinv_ref[...] = dinv[:, None]


def _tc1(xfeat, xlabel, deg2, W_gcn, W_mlp, b_mlp2):
    grid = (N // RB,)
    return pl.pallas_call(
        _tc1_body,
        grid=grid,
        in_specs=[
            pl.BlockSpec((RB, FD), lambda i: (i, 0)),
            pl.BlockSpec((RB, LD), lambda i: (i, 0)),
            pl.BlockSpec((RB, NC), lambda i: (i, 0)),
            pl.BlockSpec((FD, HD), lambda i: (0, 0)),
            pl.BlockSpec((FD + LD, HD), lambda i: (0, 0)),
            pl.BlockSpec((1, HD), lambda i: (0, 0)),
        ],
        out_specs=[
            pl.BlockSpec((RB, HD), lambda i: (i, 0)),
            pl.BlockSpec((RB, HD), lambda i: (i, 0)),
            pl.BlockSpec((RB, 1), lambda i: (i, 0)),
        ],
        out_shape=[
            jax.ShapeDtypeStruct((N, HD), jnp.float32),
            jax.ShapeDtypeStruct((N, HD), jnp.float32),
            jax.ShapeDtypeStruct((N, 1), jnp.float32),
        ],
    )(xfeat, xlabel, deg2, W_gcn, W_mlp, b_mlp2)


def _tc2_body(agg_ref, hp_ref, mlp_ref, dinv_ref, bg_ref, wc_ref, bc_ref,
              out_ref):
    agg = agg_ref[0] + hp_ref[...]
    gcn = jnp.maximum(agg * dinv_ref[...] + bg_ref[...], 0.0)
    o = jnp.dot(gcn, wc_ref[0:HD], preferred_element_type=jnp.float32)
    o = o + jnp.dot(mlp_ref[...], wc_ref[HD:2 * HD],
                    preferred_element_type=jnp.float32)
    out_ref[...] = o + bc_ref[...]


_BPH = NHALF // RB


def _tc2(aggp, hp, mlp, dinv, b_gcn2, W_cls, b_cls2):
    grid = (N // RB,)
    return pl.pallas_call(
        _tc2_body,
        grid=grid,
        in_specs=[
            pl.BlockSpec((1, RB, HD), lambda i: (i // _BPH, i % _BPH, 0)),
            pl.BlockSpec((RB, HD), lambda i: (i, 0)),
            pl.BlockSpec((RB, HD), lambda i: (i, 0)),
            pl.BlockSpec((RB, 1), lambda i: (i, 0)),
            pl.BlockSpec((1, HD), lambda i: (0, 0)),
            pl.BlockSpec((2 * HD, OD), lambda i: (0, 0)),
            pl.BlockSpec((1, OD), lambda i: (0, 0)),
        ],
        out_specs=pl.BlockSpec((RB, OD), lambda i: (i, 0)),
        out_shape=jax.ShapeDtypeStruct((N, OD), jnp.float32),
    )(aggp, hp, mlp, dinv, b_gcn2, W_cls, b_cls2)


def kernel(xfeat, xlabel, edge_index, W_gcn, b_gcn, W_mlp, b_mlp, W_cls, b_cls):
    ei = edge_index.astype(jnp.int32)
    pad = EPAD - E
    srcp = jnp.concatenate([ei[0], jnp.zeros((pad,), jnp.int32)])
    dstp = jnp.concatenate([ei[1], jnp.full((pad,), N, jnp.int32)])
    dst3 = dstp.reshape(NW, DNCH, DCH)
    src3 = srcp.reshape(NS, NCH, CH)
    dst_lo = jnp.where(dstp < NHALF, dstp, NHALF)
    dst_hi = jnp.where(dstp >= NHALF, dstp - NHALF, NHALF)
    dst4 = jnp.stack([dst_lo, dst_hi]).reshape(NC, NS, NCH, CH)
    zeros1 = jnp.zeros((RPS,), jnp.float32)
    ones1 = jnp.ones((DCH,), jnp.float32)
    zeros2 = jnp.zeros((DCH, HD), jnp.float32)

    degp = _deg_kernel(dst3, zeros1, ones1).reshape(NC, NPAD)
    deg2 = degp[:, :N].T
    hp, mlp, dinv = _tc1(xfeat, xlabel, deg2, W_gcn, W_mlp,
                         b_mlp.reshape(1, HD))
    hp_pad = jnp.pad(hp, ((0, NPAD - N), (0, 0)))
    aggp = _agg_kernel(hp_pad, src3, dst4, zeros2)
    out = _tc2(aggp, hp, mlp, dinv, b_gcn.reshape(1, HD),
               W_cls, b_cls.reshape(1, OD))
    return out
